# baseline (speedup 1.0000x reference)
import jax
import jax.numpy as jnp
from jax import lax
from jax.experimental import pallas as pl
from jax.experimental.pallas import tpu as pltpu
from jax.experimental.pallas import tpu_sc as plsc

NUM_ROWS = 100000
DIM = 64
BATCH = 4096
HIST = 50
TOTAL = BATCH * HIST
NC = 2
NS = 16
NW = NC * NS
CHUNK = 128
N_CHUNKS = TOTAL // CHUNK
PER_W = N_CHUNKS // NW
BBLK = BATCH // CHUNK


PHYS_W = CHUNK + 1


def _transpose_chunk(rows_ref, phys_ref, idx_ref, g):
    iota16 = lax.iota(jnp.int32, 16)
    d_idx = [iota16 + k * 16 for k in range(4)]
    zero16 = jnp.zeros((16,), jnp.int32)

    @plsc.parallel_loop(0, CHUNK, step=16, unroll=2)
    def _(b0):
        par16 = (idx_ref[pl.ds(g * CHUNK + b0, 16)] & 1) * DIM
        for l in range(16):
            b = b0 + l
            off = par16[l]
            bv = zero16 + b
            for k in range(4):
                v = rows_ref[b, pl.ds(off + k * 16, 16)]
                plsc.store_scatter(phys_ref, [d_idx[k], bv], v)


def _emb_body(xt_hbm, w_hbm, out_hbm, idx_v, idx2_v, rows0, rows1, ph0, ph1, g0, g1, w0, w1):
    rows = (rows0, rows1)
    phys = (ph0, ph1)
    gsem = (g0, g1)
    wsem = (w0, w1)
    wid = lax.axis_index("s") * NC + lax.axis_index("c")
    c_base = wid * PER_W
    pltpu.sync_copy(xt_hbm.at[pl.ds(c_base * CHUNK, PER_W * CHUNK)], idx_v)

    @plsc.parallel_loop(0, PER_W * CHUNK, step=16, unroll=8)
    def _(i):
        idx2_v[pl.ds(i, 16)] = lax.shift_right_logical(idx_v[pl.ds(i, 16)], 1)

    def fire_gather(g, b):
        return pltpu.async_copy(
            w_hbm.at[idx2_v.at[pl.ds(g * CHUNK, CHUNK)]], rows[b], gsem[b]
        )

    def wait_gather(g, b):
        pltpu.make_async_copy(
            w_hbm.at[idx2_v.at[pl.ds(g * CHUNK, CHUNK)]], rows[b], gsem[b]
        ).wait()

    def fire_writes(g, b):
        c = c_base + g
        t = c // BBLK
        bb = lax.rem(c, BBLK)
        for dhi in range(8):
            pltpu.async_copy(
                phys[b].at[pl.ds(dhi * 8, 8), pl.ds(0, CHUNK)],
                out_hbm.at[t, dhi, bb],
                wsem[b],
            )

    def wait_writes(b):
        for dhi in range(8):
            pltpu.make_async_copy(
                phys[b].at[pl.ds(dhi * 8, 8), pl.ds(0, CHUNK)],
                out_hbm.at[0, dhi, 0],
                wsem[b],
            ).wait()

    fire_gather(0, 0)
    fire_gather(1, 1)

    def pair_body(i, _):
        for db in range(2):
            gg = 2 * i + db
            wait_gather(gg, db)

            @pl.when(gg >= 2)
            def _():
                wait_writes(db)

            _transpose_chunk(rows[db], phys[db], idx_v, gg)
            fire_writes(gg, db)

            @pl.when(gg + 2 < PER_W)
            def _():
                fire_gather(gg + 2, db)

        return 0

    lax.fori_loop(0, PER_W // 2, pair_body, 0)
    wait_writes(0)
    wait_writes(1)


@jax.jit
def _emb(xt_flat, weight):
    mesh = plsc.VectorSubcoreMesh(
        core_axis_name="c", subcore_axis_name="s", num_cores=NC, num_subcores=NS
    )
    run = pl.kernel(
        _emb_body,
        out_type=jax.ShapeDtypeStruct((HIST, 8, BBLK, 8, CHUNK), jnp.float32),
        mesh=mesh,
        scratch_types=[
            pltpu.VMEM((PER_W * CHUNK,), jnp.int32),
            pltpu.VMEM((PER_W * CHUNK,), jnp.int32),
            pltpu.VMEM((CHUNK, 128), jnp.float32),
            pltpu.VMEM((CHUNK, 128), jnp.float32),
            pltpu.VMEM((DIM, PHYS_W), jnp.float32),
            pltpu.VMEM((DIM, PHYS_W), jnp.float32),
        ]
        + [pltpu.SemaphoreType.DMA] * 4,
        compiler_params=pltpu.CompilerParams(
            use_tc_tiling_on_sc=False, needs_layout_passes=False
        ),
    )
    return run(xt_flat, weight)


def kernel(x, weight):
    xt_flat = x.T.reshape(-1)
    wr = weight.reshape(NUM_ROWS // 2, 2 * DIM)
    phys = _emb(xt_flat, wr)
    out = phys.transpose(2, 4, 0, 1, 3).reshape(BATCH, HIST, DIM)
    return out


# speedup vs baseline: 1.4381x; 1.4381x over previous
import jax
import jax.numpy as jnp
from jax import lax
from jax.experimental import pallas as pl
from jax.experimental.pallas import tpu as pltpu
from jax.experimental.pallas import tpu_sc as plsc

NUM_ROWS = 100000
DIM = 64
BATCH = 4096
HIST = 50
TOTAL = BATCH * HIST
NC = 2
NS = 16
NW = NC * NS
CHUNK = 128
N_CHUNKS = TOTAL // CHUNK
PER_W = N_CHUNKS // NW
BBLK = BATCH // CHUNK

PHYS_W = CHUNK + 1


def _transpose_chunk(rows_ref, phys_ref):
    iota16 = lax.iota(jnp.int32, 16)
    d_idx = [iota16 + k * 16 for k in range(4)]
    zero16 = jnp.zeros((16,), jnp.int32)

    @plsc.parallel_loop(0, CHUNK, unroll=8)
    def _(b):
        bv = zero16 + b
        for k in range(4):
            v = rows_ref[b, pl.ds(k * 16, 16)]
            plsc.store_scatter(phys_ref, [d_idx[k], bv], v)


def _emb_body(xt_hbm, w_hbm, out_hbm, idx_v, rows0, rows1, ph0, ph1, g0, g1, w0, w1):
    rows = (rows0, rows1)
    phys = (ph0, ph1)
    gsem = (g0, g1)
    wsem = (w0, w1)
    wid = lax.axis_index("s") * NC + lax.axis_index("c")
    c_base = wid * PER_W
    pltpu.sync_copy(xt_hbm.at[pl.ds(c_base * CHUNK, PER_W * CHUNK)], idx_v)

    def fire_gather(g, b):
        return pltpu.async_copy(
            w_hbm.at[idx_v.at[pl.ds(g * CHUNK, CHUNK)]], rows[b], gsem[b]
        )

    def wait_gather(g, b):
        pltpu.make_async_copy(
            w_hbm.at[idx_v.at[pl.ds(g * CHUNK, CHUNK)]], rows[b], gsem[b]
        ).wait()

    def fire_writes(g, b):
        c = c_base + g
        t = c // BBLK
        bb = lax.rem(c, BBLK)
        for dhi in range(8):
            pltpu.async_copy(
                phys[b].at[pl.ds(dhi * 8, 8), pl.ds(0, CHUNK)],
                out_hbm.at[t, dhi, bb],
                wsem[b],
            )

    def wait_writes(b):
        for dhi in range(8):
            pltpu.make_async_copy(
                phys[b].at[pl.ds(dhi * 8, 8), pl.ds(0, CHUNK)],
                out_hbm.at[0, dhi, 0],
                wsem[b],
            ).wait()

    fire_gather(0, 0)
    fire_gather(1, 1)

    def pair_body(i, _):
        for db in range(2):
            gg = 2 * i + db
            wait_gather(gg, db)

            @pl.when(gg >= 2)
            def _():
                wait_writes(db)

            _transpose_chunk(rows[db], phys[db])
            fire_writes(gg, db)

            @pl.when(gg + 2 < PER_W)
            def _():
                fire_gather(gg + 2, db)

        return 0

    lax.fori_loop(0, PER_W // 2, pair_body, 0)
    wait_writes(0)
    wait_writes(1)


@jax.jit
def _emb(xt_flat, weight):
    mesh = plsc.VectorSubcoreMesh(
        core_axis_name="c", subcore_axis_name="s", num_cores=NC, num_subcores=NS
    )
    run = pl.kernel(
        _emb_body,
        out_type=jax.ShapeDtypeStruct((HIST, 8, BBLK, 8, CHUNK), jnp.float32),
        mesh=mesh,
        scratch_types=[
            pltpu.VMEM((PER_W * CHUNK,), jnp.int32),
            pltpu.VMEM((CHUNK, DIM), jnp.float32),
            pltpu.VMEM((CHUNK, DIM), jnp.float32),
            pltpu.VMEM((DIM, PHYS_W), jnp.float32),
            pltpu.VMEM((DIM, PHYS_W), jnp.float32),
        ]
        + [pltpu.SemaphoreType.DMA] * 4,
        compiler_params=pltpu.CompilerParams(
            use_tc_tiling_on_sc=False, needs_layout_passes=False
        ),
    )
    return run(xt_flat, weight)


def kernel(x, weight):
    xt_flat = x.T.reshape(-1)
    phys = _emb(xt_flat, weight)
    out = phys.transpose(2, 4, 0, 1, 3).reshape(BATCH, HIST, DIM)
    return out
